# Initial kernel scaffold; baseline (speedup 1.0000x reference)
import jax
import jax.numpy as jnp
from jax.experimental import pallas as pl


def kernel(x, edge_index, W1a, b1a, W1b, b1b, W2a, b2a, W2b, b2b, Wo, bo):
    raise NotImplementedError("write your pallas kernel here")


# speedup vs baseline: 5.8601x; 5.8601x over previous
import functools

import jax
import jax.numpy as jnp
from jax import lax
from jax.experimental import pallas as pl
from jax.experimental.pallas import tpu as pltpu
from jax.experimental.pallas import tpu_sc as plsc

_NC = 2
_NS = 16
_NW = _NC * _NS
_CH = 128
_ROW_BLK = 1024


def _matmul_body(x_ref, w_ref, o_ref):
    o_ref[...] = jnp.dot(x_ref[...], w_ref[...], preferred_element_type=jnp.float32, precision=lax.Precision.HIGHEST)


def _mlp_mid_body(t_ref, p0_ref, p1_ref, b1a_ref, w1b_ref, b1b_ref, w2a_ref, o_ref):
    a = jnp.maximum(t_ref[...] + p0_ref[...] + p1_ref[...] + b1a_ref[...], 0.0)
    h1 = jnp.dot(a, w1b_ref[...], preferred_element_type=jnp.float32, precision=lax.Precision.HIGHEST) + b1b_ref[...]
    o_ref[...] = jnp.dot(h1, w2a_ref[...], preferred_element_type=jnp.float32, precision=lax.Precision.HIGHEST)


def _mlp_out_body(u_ref, q0_ref, q1_ref, b2a_ref, w2b_ref, b2b_ref, wo_ref, bo_ref,
                  o_ref):
    a = jnp.maximum(u_ref[...] + q0_ref[...] + q1_ref[...] + b2a_ref[...], 0.0)
    h2 = jnp.dot(a, w2b_ref[...], preferred_element_type=jnp.float32, precision=lax.Precision.HIGHEST) + b2b_ref[...]
    o_ref[...] = jnp.sum(h2 * wo_ref[...], axis=1, keepdims=True) + bo_ref[...]


@functools.lru_cache(maxsize=None)
def _make_sc_agg(np_, h, nchw):
    rpt = np_ // _NS
    nzc = rpt // _CH
    mesh = plsc.VectorSubcoreMesh(core_axis_name="c", subcore_axis_name="s")

    @functools.partial(
        pl.kernel,
        out_type=jax.ShapeDtypeStruct((_NC, np_, h), jnp.float32),
        mesh=mesh,
        scratch_types=[
            pltpu.VMEM((nchw, _CH), jnp.int32),
            pltpu.VMEM((nchw, _CH), jnp.int32),
            pltpu.VMEM((_CH, h), jnp.float32),
            pltpu.VMEM_SHARED((np_, h), jnp.float32),
            pltpu.SemaphoreType.DMA,
        ],
        compiler_params=pltpu.CompilerParams(use_tc_tiling_on_sc=False),
    )
    def agg(t_hbm, src_hbm, dst_hbm, zero_hbm, out_hbm, src_v, dst_v, rows_v,
            acc_sh, sem):
        c = lax.axis_index("c")
        s = lax.axis_index("s")
        wid = c * _NS + s
        base = s * rpt
        pltpu.sync_copy(zero_hbm, rows_v)
        for k in range(nzc):
            pltpu.sync_copy(rows_v, acc_sh.at[pl.ds(base + k * _CH, _CH)])
        pltpu.sync_copy(src_hbm.at[wid], src_v)
        pltpu.sync_copy(dst_hbm.at[wid], dst_v)
        plsc.subcore_barrier()

        def chunk(j, carry):
            pltpu.async_copy(t_hbm.at[src_v.at[j]], rows_v, sem).wait()
            pltpu.sync_copy(rows_v, acc_sh.at[dst_v.at[j]], add=True)
            return carry

        lax.fori_loop(0, nchw, chunk, 0)
        plsc.subcore_barrier()
        for k in range(nzc):
            pltpu.sync_copy(acc_sh.at[pl.ds(base + k * _CH, _CH)], rows_v)
            pltpu.sync_copy(rows_v, out_hbm.at[c, pl.ds(base + k * _CH, _CH)])

    return agg


def _blk(shp):
    return pl.BlockSpec(shp, lambda i: (i, 0))


def _whole(shp):
    return pl.BlockSpec(shp, lambda i: (0, 0))


def kernel(x, edge_index, W1a, b1a, W1b, b1b, W2a, b2a, W2b, b2b, Wo, bo):
    n, d = x.shape
    h = W1a.shape[1]
    e = edge_index.shape[1]
    np_ = ((n + 1 + _ROW_BLK - 1) // _ROW_BLK) * _ROW_BLK
    nblk = np_ // _ROW_BLK
    nchw = -(-e // (_NW * _CH))
    e_pad = _NW * _CH * nchw

    fill = jnp.full((e_pad - e,), n, jnp.int32)
    src = jnp.concatenate([edge_index[0], fill]).reshape(_NW, nchw, _CH)
    dst = jnp.concatenate([edge_index[1], fill]).reshape(_NW, nchw, _CH)
    x_pad = jnp.pad(x, ((0, np_ - n), (0, 0)))
    zeros = jnp.zeros((_CH, h), jnp.float32)

    b1a_r, b1b_r, b2a_r, b2b_r = (v.reshape(1, h) for v in (b1a, b1b, b2a, b2b))
    wo_r = Wo.reshape(1, h)
    bo_r = bo.reshape(1, 1)

    t = pl.pallas_call(
        _matmul_body,
        grid=(nblk,),
        in_specs=[_blk((_ROW_BLK, d)), _whole((d, h))],
        out_specs=_blk((_ROW_BLK, h)),
        out_shape=jax.ShapeDtypeStruct((np_, h), jnp.float32),
    )(x_pad, W1a)

    sc_agg = _make_sc_agg(np_, h, nchw)

    p = sc_agg(t, src, dst, zeros)

    u = pl.pallas_call(
        _mlp_mid_body,
        grid=(nblk,),
        in_specs=[_blk((_ROW_BLK, h)), _blk((_ROW_BLK, h)), _blk((_ROW_BLK, h)),
                  _whole((1, h)), _whole((h, h)), _whole((1, h)), _whole((h, h))],
        out_specs=_blk((_ROW_BLK, h)),
        out_shape=jax.ShapeDtypeStruct((np_, h), jnp.float32),
    )(t, p[0], p[1], b1a_r, W1b, b1b_r, W2a)

    q = sc_agg(u, src, dst, zeros)

    res = pl.pallas_call(
        _mlp_out_body,
        grid=(nblk,),
        in_specs=[_blk((_ROW_BLK, h)), _blk((_ROW_BLK, h)), _blk((_ROW_BLK, h)),
                  _whole((1, h)), _whole((h, h)), _whole((1, h)), _whole((1, h)),
                  _whole((1, 1))],
        out_specs=_blk((_ROW_BLK, 1)),
        out_shape=jax.ShapeDtypeStruct((np_, 1), jnp.float32),
    )(u, q[0], q[1], b2a_r, W2b, b2b_r, wo_r, bo_r)

    return res[:n, 0]
